# baseline (speedup 1.0000x reference)
import jax
import jax.numpy as jnp
import numpy as _np
from jax import lax
from jax.experimental import pallas as pl
from jax.experimental.pallas import tpu as pltpu
from jax.experimental.pallas import tpu_sc as plsc

B = 16384
D = 64
NP = 1000
NU = 100
NC = 2
NS = 16
NW = NC * NS
C = B // NW
RPT = 32
NG = C // 16


def _rsqrt_nr(s):
    i = plsc.bitcast(s, jnp.int32)
    i = jnp.int32(0x5F3759DF) - (i >> 1)
    y = plsc.bitcast(i, jnp.float32)
    for _ in range(3):
        y = y * (1.5 - ((0.5 * s) * y) * y)
    return y


def _sqrt_nr(s):
    return s * _rsqrt_nr(s)


def _body(user_hbm, prev_hbm, pos_hbm, neg_hbm, poi_hbm, usr_hbm, g_hbm,
          bias_hbm, obj_hbm, w_hbm,
          poi_v, usr_v, bias_v, g_v, uid_v, pid_v, sid_v, nid_v,
          obj_v, wout_v, poi_sh, sem, sem2):
    wid = lax.axis_index("s") * NC + lax.axis_index("c")
    base = wid * C

    with jax.named_scope("stage_dma"):
        sid = lax.axis_index("s")
        chunk = NP * D // NS
        soff = pl.multiple_of(sid * chunk, 8)
        dmas = [
            pltpu.async_copy(usr_hbm, usr_v, sem),
            pltpu.async_copy(bias_hbm, bias_v, sem),
            pltpu.async_copy(g_hbm, g_v, sem),
            pltpu.async_copy(user_hbm.at[pl.ds(base, C)], uid_v, sem),
            pltpu.async_copy(prev_hbm.at[pl.ds(base, C)], pid_v, sem),
            pltpu.async_copy(pos_hbm.at[pl.ds(base, C)], sid_v, sem),
            pltpu.async_copy(neg_hbm.at[pl.ds(base, C)], nid_v, sem),
        ]
        pltpu.sync_copy(poi_hbm.at[pl.ds(soff, chunk)],
                        poi_v.at[pl.ds(soff, chunk)])
        pltpu.sync_copy(poi_v.at[pl.ds(soff, chunk)],
                        poi_sh.at[pl.ds(soff, chunk)])
        plsc.subcore_barrier()
        poi_dma = pltpu.async_copy(poi_sh, poi_v, sem2)
        for d in dmas:
            d.wait()

    g0 = g_v[pl.ds(0, 16)]
    g1 = g_v[pl.ds(16, 16)]
    g2v = g_v[pl.ds(32, 16)]
    g3 = g_v[pl.ds(48, 16)]

    def fold(r, gs):
        ro = pl.multiple_of(r * D, 16)
        for q in range(4):
            off_q = ro + q * 16
            usr_v[pl.ds(off_q, 16)] = usr_v[pl.ds(off_q, 16)] + gs[q]
        return gs

    lax.fori_loop(0, NU, fold, (g0, g1, g2v, g3))
    poi_dma.wait()

    def group(g):
        off = pl.multiple_of(g * 16, 16)
        up = uid_v[pl.ds(off, 16)] * D
        pp = pid_v[pl.ds(off, 16)] * D
        sp = sid_v[pl.ds(off, 16)]
        sn = nid_v[pl.ds(off, 16)]
        bp = plsc.load_gather(bias_v, [sp])
        bn = plsc.load_gather(bias_v, [sn])
        sp = sp * D
        sn = sn * D
        acc_p = jnp.zeros((16,), jnp.float32)
        acc_n = jnp.zeros((16,), jnp.float32)
        lane = lax.iota(jnp.int32, 16)
        for j in range(D):
            jr = (lane + j) & (D - 1)
            t = (plsc.load_gather(poi_v, [pp + jr])
                 + plsc.load_gather(usr_v, [up + jr]))
            ep = t - plsc.load_gather(poi_v, [sp + jr])
            en = t - plsc.load_gather(poi_v, [sn + jr])
            acc_p = acc_p + ep * ep
            acc_n = acc_n + en * en
        obj = (bp - bn) + _sqrt_nr(acc_n) - _sqrt_nr(acc_p)
        obj_v[pl.ds(off, 16)] = obj

    with jax.named_scope("main_loop"):
        plsc.parallel_loop(0, NG, 1, unroll=4)(group)
    obj_dma = pltpu.async_copy(obj_v, obj_hbm.at[pl.ds(base, C)], sem)

    base_r = wid * RPT
    iota = lax.iota(jnp.int32, 16)
    for g2 in range(RPT // 16):
        ridx = jnp.minimum(base_r + g2 * 16 + iota, NP - 1) * D

        def nsum(j, acc):
            jr = (iota + j) & (D - 1)
            v = plsc.load_gather(poi_v, [ridx + jr])
            return acc + v * v

        s = lax.fori_loop(0, D, nsum, jnp.zeros((16,), jnp.float32))
        scale = jnp.minimum(1.0, _rsqrt_nr(s))
        lrow = (g2 * 16 + iota) * D

        def nwrite(j, carry):
            jr = (iota + j) & (D - 1)
            v = plsc.load_gather(poi_v, [ridx + jr])
            plsc.store_scatter(wout_v, [lrow + jr], v * scale)
            return carry

        lax.fori_loop(0, D, nwrite, None)

    @pl.when(wid < NW - 1)
    def _():
        pltpu.async_copy(wout_v, w_hbm.at[pl.ds(base_r * D, RPT * D)],
                         sem2).wait()

    @pl.when(wid == NW - 1)
    def _():
        rem = NP - (NW - 1) * RPT
        pltpu.async_copy(wout_v.at[pl.ds(0, rem * D)],
                         w_hbm.at[pl.ds((NW - 1) * RPT * D, rem * D)],
                         sem2).wait()

    obj_dma.wait()


@jax.jit
def kernel(user_id, prev_id, pos_id, neg_id, poi_weight, user_weight,
           user_global_weight, poi_bias_weight):
    mesh = plsc.VectorSubcoreMesh(core_axis_name="c", subcore_axis_name="s")
    prog = pl.kernel(
        _body,
        out_type=(
            jax.ShapeDtypeStruct((B,), jnp.float32),
            jax.ShapeDtypeStruct((NP * D,), jnp.float32),
        ),
        mesh=mesh,
        compiler_params=pltpu.CompilerParams(
            needs_layout_passes=False,
            disable_bounds_checks=True,
            skip_device_barrier=True,
        ),
        scratch_types=[
            pltpu.VMEM((NP * D,), jnp.float32),
            pltpu.VMEM((NU * D,), jnp.float32),
            pltpu.VMEM((NP,), jnp.float32),
            pltpu.VMEM((D,), jnp.float32),
            pltpu.VMEM((C,), jnp.int32),
            pltpu.VMEM((C,), jnp.int32),
            pltpu.VMEM((C,), jnp.int32),
            pltpu.VMEM((C,), jnp.int32),
            pltpu.VMEM((C,), jnp.float32),
            pltpu.VMEM((RPT * D,), jnp.float32),
            pltpu.VMEM_SHARED((NP * D,), jnp.float32),
            pltpu.SemaphoreType.DMA,
            pltpu.SemaphoreType.DMA,
        ],
    )
    obj, w = prog(
        user_id.astype(jnp.int32),
        prev_id.astype(jnp.int32),
        pos_id.astype(jnp.int32),
        neg_id.astype(jnp.int32),
        poi_weight.reshape(NP * D),
        user_weight.reshape(NU * D),
        user_global_weight.reshape(D),
        poi_bias_weight.reshape(NP),
    )
    return obj, w.reshape(NP, D)


# speedup vs baseline: 1.2254x; 1.2254x over previous
import jax
import jax.numpy as jnp
import numpy as _np
from jax import lax
from jax.experimental import pallas as pl
from jax.experimental.pallas import tpu as pltpu
from jax.experimental.pallas import tpu_sc as plsc

B = 16384
D = 64
NP = 1000
NU = 100
NC = 2
NS = 16
NW = NC * NS
C = B // NW
RPT = 64
NG = C // 16


def _rsqrt_nr(s):
    i = plsc.bitcast(s, jnp.int32)
    i = jnp.int32(0x5F3759DF) - (i >> 1)
    y = plsc.bitcast(i, jnp.float32)
    for _ in range(3):
        y = y * (1.5 - ((0.5 * s) * y) * y)
    return y


def _sqrt_nr(s):
    return s * _rsqrt_nr(s)


def _body(user_hbm, prev_hbm, pos_hbm, neg_hbm, poi_hbm, usr_hbm, g_hbm,
          bias_hbm, obj_hbm, w_hbm,
          poi_v, usr_v, bias_v, g_v, uid_v, pid_v, sid_v, nid_v,
          obj_v, wout_v, poi_sh, sem, sem2):
    wid = lax.axis_index("s") * NC + lax.axis_index("c")
    base = wid * C

    with jax.named_scope("stage_dma"):
        sid = lax.axis_index("s")
        soff = pl.multiple_of(sid * RPT * D, 8)
        dmas = [
            pltpu.async_copy(usr_hbm, usr_v, sem),
            pltpu.async_copy(bias_hbm, bias_v, sem),
            pltpu.async_copy(g_hbm, g_v, sem),
            pltpu.async_copy(user_hbm.at[pl.ds(base, C)], uid_v, sem),
            pltpu.async_copy(prev_hbm.at[pl.ds(base, C)], pid_v, sem),
            pltpu.async_copy(pos_hbm.at[pl.ds(base, C)], sid_v, sem),
            pltpu.async_copy(neg_hbm.at[pl.ds(base, C)], nid_v, sem),
        ]
        full = RPT * D
        part = (NP - (NS - 1) * RPT) * D

        @pl.when(sid < NS - 1)
        def _():
            pltpu.sync_copy(poi_hbm.at[pl.ds(soff, full)],
                            poi_v.at[pl.ds(soff, full)])
            pltpu.sync_copy(poi_v.at[pl.ds(soff, full)],
                            poi_sh.at[pl.ds(soff, full)])

        @pl.when(sid == NS - 1)
        def _():
            pltpu.sync_copy(poi_hbm.at[pl.ds(soff, part)],
                            poi_v.at[pl.ds(soff, part)])
            pltpu.sync_copy(poi_v.at[pl.ds(soff, part)],
                            poi_sh.at[pl.ds(soff, part)])

        plsc.subcore_barrier()
        poi_dma = pltpu.async_copy(poi_sh, poi_v, sem2)
        for d in dmas:
            d.wait()

    g0 = g_v[pl.ds(0, 16)]
    g1 = g_v[pl.ds(16, 16)]
    g2v = g_v[pl.ds(32, 16)]
    g3 = g_v[pl.ds(48, 16)]

    def fold(r, gs):
        ro = pl.multiple_of(r * D, 16)
        for q in range(4):
            off_q = ro + q * 16
            usr_v[pl.ds(off_q, 16)] = usr_v[pl.ds(off_q, 16)] + gs[q]
        return gs

    lax.fori_loop(0, NU, fold, (g0, g1, g2v, g3))

    base_r = sid * RPT
    iota = lax.iota(jnp.int32, 16)
    for g2 in range(RPT // 16):
        ridx = jnp.minimum(base_r + g2 * 16 + iota, NP - 1) * D

        def nsum(j, acc):
            jr = (iota + j) & (D - 1)
            v = plsc.load_gather(poi_v, [ridx + jr])
            return acc + v * v

        s = lax.fori_loop(0, D, nsum, jnp.zeros((16,), jnp.float32))
        scale = jnp.minimum(1.0, _rsqrt_nr(s))
        lrow = (g2 * 16 + iota) * D

        def nwrite(j, carry):
            jr = (iota + j) & (D - 1)
            v = plsc.load_gather(poi_v, [ridx + jr])
            plsc.store_scatter(wout_v, [lrow + jr], v * scale)
            return carry

        lax.fori_loop(0, D, nwrite, None)

    @pl.when(sid < NS - 1)
    def _():
        pltpu.async_copy(wout_v, w_hbm.at[pl.ds(soff, full)], sem)

    @pl.when(sid == NS - 1)
    def _():
        pltpu.async_copy(wout_v.at[pl.ds(0, part)],
                         w_hbm.at[pl.ds(soff, part)], sem)

    poi_dma.wait()

    def group(g):
        off = pl.multiple_of(g * 16, 16)
        up = uid_v[pl.ds(off, 16)] * D
        pp = pid_v[pl.ds(off, 16)] * D
        sp = sid_v[pl.ds(off, 16)]
        sn = nid_v[pl.ds(off, 16)]
        bp = plsc.load_gather(bias_v, [sp])
        bn = plsc.load_gather(bias_v, [sn])
        sp = sp * D
        sn = sn * D
        acc_p = jnp.zeros((16,), jnp.float32)
        acc_n = jnp.zeros((16,), jnp.float32)
        lane = lax.iota(jnp.int32, 16)
        for j in range(D):
            jr = (lane + j) & (D - 1)
            t = (plsc.load_gather(poi_v, [pp + jr])
                 + plsc.load_gather(usr_v, [up + jr]))
            ep = t - plsc.load_gather(poi_v, [sp + jr])
            en = t - plsc.load_gather(poi_v, [sn + jr])
            acc_p = acc_p + ep * ep
            acc_n = acc_n + en * en
        obj = (bp - bn) + _sqrt_nr(acc_n) - _sqrt_nr(acc_p)
        obj_v[pl.ds(off, 16)] = obj

    with jax.named_scope("main_loop"):
        plsc.parallel_loop(0, NG, 1, unroll=2)(group)
    obj_dma = pltpu.async_copy(obj_v, obj_hbm.at[pl.ds(base, C)], sem)

    @pl.when(sid < NS - 1)
    def _():
        pltpu.make_async_copy(wout_v, w_hbm.at[pl.ds(soff, full)],
                              sem).wait()

    @pl.when(sid == NS - 1)
    def _():
        pltpu.make_async_copy(wout_v.at[pl.ds(0, part)],
                              w_hbm.at[pl.ds(soff, part)], sem).wait()

    obj_dma.wait()


@jax.jit
def kernel(user_id, prev_id, pos_id, neg_id, poi_weight, user_weight,
           user_global_weight, poi_bias_weight):
    mesh = plsc.VectorSubcoreMesh(core_axis_name="c", subcore_axis_name="s")
    prog = pl.kernel(
        _body,
        out_type=(
            jax.ShapeDtypeStruct((B,), jnp.float32),
            jax.ShapeDtypeStruct((NP * D,), jnp.float32),
        ),
        mesh=mesh,
        compiler_params=pltpu.CompilerParams(
            needs_layout_passes=False,
            disable_bounds_checks=True,
        ),
        scratch_types=[
            pltpu.VMEM((NP * D,), jnp.float32),
            pltpu.VMEM((NU * D,), jnp.float32),
            pltpu.VMEM((NP,), jnp.float32),
            pltpu.VMEM((D,), jnp.float32),
            pltpu.VMEM((C,), jnp.int32),
            pltpu.VMEM((C,), jnp.int32),
            pltpu.VMEM((C,), jnp.int32),
            pltpu.VMEM((C,), jnp.int32),
            pltpu.VMEM((C,), jnp.float32),
            pltpu.VMEM((RPT * D,), jnp.float32),
            pltpu.VMEM_SHARED((NP * D,), jnp.float32),
            pltpu.SemaphoreType.DMA,
            pltpu.SemaphoreType.DMA,
        ],
    )
    obj, w = prog(
        user_id.astype(jnp.int32),
        prev_id.astype(jnp.int32),
        pos_id.astype(jnp.int32),
        neg_id.astype(jnp.int32),
        poi_weight.reshape(NP * D),
        user_weight.reshape(NU * D),
        user_global_weight.reshape(D),
        poi_bias_weight.reshape(NP),
    )
    return obj, w.reshape(NP, D)


# speedup vs baseline: 1.2360x; 1.0087x over previous
import jax
import jax.numpy as jnp
import numpy as _np
from jax import lax
from jax.experimental import pallas as pl
from jax.experimental.pallas import tpu as pltpu
from jax.experimental.pallas import tpu_sc as plsc

B = 16384
D = 64
NP = 1000
NU = 100
NC = 2
NS = 16
NW = NC * NS
C = B // NW
RPT = 64
NG = C // 16


def _rsqrt_nr(s):
    i = plsc.bitcast(s, jnp.int32)
    i = jnp.int32(0x5F3759DF) - (i >> 1)
    y = plsc.bitcast(i, jnp.float32)
    for _ in range(3):
        y = y * (1.5 - ((0.5 * s) * y) * y)
    return y


def _sqrt_nr(s):
    return s * _rsqrt_nr(s)


def _body(user_hbm, prev_hbm, pos_hbm, neg_hbm, poi_hbm, usr_hbm, g_hbm,
          bias_hbm, obj_hbm, w_hbm,
          poi_v, usr_v, bias_v, g_v, uid_v, pid_v, sid_v, nid_v,
          obj_v, wout_v, poi_sh, sem, sem2):
    wid = lax.axis_index("s") * NC + lax.axis_index("c")
    base = wid * C

    with jax.named_scope("stage_dma"):
        sid = lax.axis_index("s")
        soff = pl.multiple_of(sid * RPT * D, 8)
        dmas = [
            pltpu.async_copy(usr_hbm, usr_v, sem),
            pltpu.async_copy(bias_hbm, bias_v, sem),
            pltpu.async_copy(g_hbm, g_v, sem),
            pltpu.async_copy(user_hbm.at[pl.ds(base, C)], uid_v, sem),
            pltpu.async_copy(prev_hbm.at[pl.ds(base, C)], pid_v, sem),
            pltpu.async_copy(pos_hbm.at[pl.ds(base, C)], sid_v, sem),
            pltpu.async_copy(neg_hbm.at[pl.ds(base, C)], nid_v, sem),
        ]
        full = RPT * D
        part = (NP - (NS - 1) * RPT) * D

        @pl.when(sid < NS - 1)
        def _():
            pltpu.sync_copy(poi_hbm.at[pl.ds(soff, full)],
                            poi_v.at[pl.ds(soff, full)])
            pltpu.sync_copy(poi_v.at[pl.ds(soff, full)],
                            poi_sh.at[pl.ds(soff, full)])

        @pl.when(sid == NS - 1)
        def _():
            pltpu.sync_copy(poi_hbm.at[pl.ds(soff, part)],
                            poi_v.at[pl.ds(soff, part)])
            pltpu.sync_copy(poi_v.at[pl.ds(soff, part)],
                            poi_sh.at[pl.ds(soff, part)])

        plsc.subcore_barrier()
        poi_dma = pltpu.async_copy(poi_sh, poi_v, sem2)
        for d in dmas:
            d.wait()

    g0 = g_v[pl.ds(0, 16)]
    g1 = g_v[pl.ds(16, 16)]
    g2v = g_v[pl.ds(32, 16)]
    g3 = g_v[pl.ds(48, 16)]

    def fold(r, gs):
        ro = pl.multiple_of(r * D, 16)
        for q in range(4):
            off_q = ro + q * 16
            usr_v[pl.ds(off_q, 16)] = usr_v[pl.ds(off_q, 16)] + gs[q]
        return gs

    lax.fori_loop(0, NU, fold, (g0, g1, g2v, g3))

    base_r = sid * RPT
    iota = lax.iota(jnp.int32, 16)
    for g2 in range(RPT // 16):
        ridx = jnp.minimum(base_r + g2 * 16 + iota, NP - 1) * D

        def nsum(j, acc):
            jr = (iota + j) & (D - 1)
            v = plsc.load_gather(poi_v, [ridx + jr])
            return acc + v * v

        s = lax.fori_loop(0, D, nsum, jnp.zeros((16,), jnp.float32))
        scale = jnp.minimum(1.0, _rsqrt_nr(s))
        lrow = (g2 * 16 + iota) * D

        def nwrite(j, carry):
            jr = (iota + j) & (D - 1)
            v = plsc.load_gather(poi_v, [ridx + jr])
            plsc.store_scatter(wout_v, [lrow + jr], v * scale)
            return carry

        lax.fori_loop(0, D, nwrite, None)

    @pl.when(sid < NS - 1)
    def _():
        pltpu.async_copy(wout_v, w_hbm.at[pl.ds(soff, full)], sem)

    @pl.when(sid == NS - 1)
    def _():
        pltpu.async_copy(wout_v.at[pl.ds(0, part)],
                         w_hbm.at[pl.ds(soff, part)], sem)

    poi_dma.wait()

    def group(g):
        off = pl.multiple_of(g * 16, 16)
        up = uid_v[pl.ds(off, 16)] * D
        pp = pid_v[pl.ds(off, 16)] * D
        sp = sid_v[pl.ds(off, 16)]
        sn = nid_v[pl.ds(off, 16)]
        bp = plsc.load_gather(bias_v, [sp])
        bn = plsc.load_gather(bias_v, [sn])
        sp = sp * D
        sn = sn * D
        zero = jnp.zeros((16,), jnp.float32)
        ap = [zero] * 4
        an = [zero] * 4
        lane = lax.iota(jnp.int32, 16)
        for j in range(D):
            jr = (lane + j) & (D - 1)
            t = (plsc.load_gather(poi_v, [pp + jr])
                 + plsc.load_gather(usr_v, [up + jr]))
            ep = t - plsc.load_gather(poi_v, [sp + jr])
            en = t - plsc.load_gather(poi_v, [sn + jr])
            q = j & 3
            ap[q] = ap[q] + ep * ep
            an[q] = an[q] + en * en
        acc_p = (ap[0] + ap[1]) + (ap[2] + ap[3])
        acc_n = (an[0] + an[1]) + (an[2] + an[3])
        obj = (bp - bn) + _sqrt_nr(acc_n) - _sqrt_nr(acc_p)
        obj_v[pl.ds(off, 16)] = obj

    with jax.named_scope("main_loop"):
        plsc.parallel_loop(0, NG, 1, unroll=2)(group)
    obj_dma = pltpu.async_copy(obj_v, obj_hbm.at[pl.ds(base, C)], sem)

    @pl.when(sid < NS - 1)
    def _():
        pltpu.make_async_copy(wout_v, w_hbm.at[pl.ds(soff, full)],
                              sem).wait()

    @pl.when(sid == NS - 1)
    def _():
        pltpu.make_async_copy(wout_v.at[pl.ds(0, part)],
                              w_hbm.at[pl.ds(soff, part)], sem).wait()

    obj_dma.wait()


@jax.jit
def kernel(user_id, prev_id, pos_id, neg_id, poi_weight, user_weight,
           user_global_weight, poi_bias_weight):
    mesh = plsc.VectorSubcoreMesh(core_axis_name="c", subcore_axis_name="s")
    prog = pl.kernel(
        _body,
        out_type=(
            jax.ShapeDtypeStruct((B,), jnp.float32),
            jax.ShapeDtypeStruct((NP * D,), jnp.float32),
        ),
        mesh=mesh,
        compiler_params=pltpu.CompilerParams(
            needs_layout_passes=False,
            disable_bounds_checks=True,
        ),
        scratch_types=[
            pltpu.VMEM((NP * D,), jnp.float32),
            pltpu.VMEM((NU * D,), jnp.float32),
            pltpu.VMEM((NP,), jnp.float32),
            pltpu.VMEM((D,), jnp.float32),
            pltpu.VMEM((C,), jnp.int32),
            pltpu.VMEM((C,), jnp.int32),
            pltpu.VMEM((C,), jnp.int32),
            pltpu.VMEM((C,), jnp.int32),
            pltpu.VMEM((C,), jnp.float32),
            pltpu.VMEM((RPT * D,), jnp.float32),
            pltpu.VMEM_SHARED((NP * D,), jnp.float32),
            pltpu.SemaphoreType.DMA,
            pltpu.SemaphoreType.DMA,
        ],
    )
    obj, w = prog(
        user_id.astype(jnp.int32),
        prev_id.astype(jnp.int32),
        pos_id.astype(jnp.int32),
        neg_id.astype(jnp.int32),
        poi_weight.reshape(NP * D),
        user_weight.reshape(NU * D),
        user_global_weight.reshape(D),
        poi_bias_weight.reshape(NP),
    )
    return obj, w.reshape(NP, D)


# speedup vs baseline: 1.2435x; 1.0061x over previous
import jax
import jax.numpy as jnp
import numpy as _np
from jax import lax
from jax.experimental import pallas as pl
from jax.experimental.pallas import tpu as pltpu
from jax.experimental.pallas import tpu_sc as plsc

B = 16384
D = 64
NP = 1000
NU = 100
NC = 2
NS = 16
NW = NC * NS
C = B // NW
RPT = 64
NG = C // 16


def _rsqrt_nr(s):
    i = plsc.bitcast(s, jnp.int32)
    i = jnp.int32(0x5F3759DF) - (i >> 1)
    y = plsc.bitcast(i, jnp.float32)
    for _ in range(3):
        y = y * (1.5 - ((0.5 * s) * y) * y)
    return y


def _sqrt_nr(s):
    return s * _rsqrt_nr(s)


def _body(user_hbm, prev_hbm, pos_hbm, neg_hbm, poi_hbm, usr_hbm, g_hbm,
          bias_hbm, obj_hbm, w_hbm,
          poi_v, usr_v, bias_v, g_v, uid_v, pid_v, sid_v, nid_v,
          obj_v, wout_v, poi_sh, sem, sem2):
    wid = lax.axis_index("s") * NC + lax.axis_index("c")
    base = wid * C

    with jax.named_scope("stage_dma"):
        sid = lax.axis_index("s")
        soff = pl.multiple_of(sid * RPT * D, 8)
        dmas = [
            pltpu.async_copy(usr_hbm, usr_v, sem),
            pltpu.async_copy(bias_hbm, bias_v, sem),
            pltpu.async_copy(g_hbm, g_v, sem),
            pltpu.async_copy(user_hbm.at[pl.ds(base, C)], uid_v, sem),
            pltpu.async_copy(prev_hbm.at[pl.ds(base, C)], pid_v, sem),
            pltpu.async_copy(pos_hbm.at[pl.ds(base, C)], sid_v, sem),
            pltpu.async_copy(neg_hbm.at[pl.ds(base, C)], nid_v, sem),
        ]
        full = RPT * D
        part = (NP - (NS - 1) * RPT) * D

        @pl.when(sid < NS - 1)
        def _():
            pltpu.sync_copy(poi_hbm.at[pl.ds(soff, full)],
                            poi_v.at[pl.ds(soff, full)])
            pltpu.sync_copy(poi_v.at[pl.ds(soff, full)],
                            poi_sh.at[pl.ds(soff, full)])

        @pl.when(sid == NS - 1)
        def _():
            pltpu.sync_copy(poi_hbm.at[pl.ds(soff, part)],
                            poi_v.at[pl.ds(soff, part)])
            pltpu.sync_copy(poi_v.at[pl.ds(soff, part)],
                            poi_sh.at[pl.ds(soff, part)])

        plsc.subcore_barrier()
        poi_dma = pltpu.async_copy(poi_sh, poi_v, sem2)
        for d in dmas:
            d.wait()

    g0 = g_v[pl.ds(0, 16)]
    g1 = g_v[pl.ds(16, 16)]
    g2v = g_v[pl.ds(32, 16)]
    g3 = g_v[pl.ds(48, 16)]

    def fold(r, gs):
        ro = pl.multiple_of(r * D, 16)
        for q in range(4):
            off_q = ro + q * 16
            usr_v[pl.ds(off_q, 16)] = usr_v[pl.ds(off_q, 16)] + gs[q]
        return gs

    lax.fori_loop(0, NU, fold, (g0, g1, g2v, g3))

    base_r = sid * RPT
    iota = lax.iota(jnp.int32, 16)
    for g2 in range(RPT // 16):
        ridx = jnp.minimum(base_r + g2 * 16 + iota, NP - 1) * D

        def nsum(j, acc):
            jr = (iota + j) & (D - 1)
            v = plsc.load_gather(poi_v, [ridx + jr])
            return acc + v * v

        s = lax.fori_loop(0, D, nsum, jnp.zeros((16,), jnp.float32))
        scale = jnp.minimum(1.0, _rsqrt_nr(s))
        lrow = (g2 * 16 + iota) * D

        def nwrite(j, carry):
            jr = (iota + j) & (D - 1)
            v = plsc.load_gather(poi_v, [ridx + jr])
            plsc.store_scatter(wout_v, [lrow + jr], v * scale)
            return carry

        lax.fori_loop(0, D, nwrite, None)

    def biasg(g, carry):
        off = pl.multiple_of(g * 16, 16)
        bp = plsc.load_gather(bias_v, [sid_v[pl.ds(off, 16)]])
        bn = plsc.load_gather(bias_v, [nid_v[pl.ds(off, 16)]])
        obj_v[pl.ds(off, 16)] = bp - bn
        return carry

    lax.fori_loop(0, NG, biasg, None)

    @pl.when(sid < NS - 1)
    def _():
        pltpu.async_copy(wout_v, w_hbm.at[pl.ds(soff, full)], sem)

    @pl.when(sid == NS - 1)
    def _():
        pltpu.async_copy(wout_v.at[pl.ds(0, part)],
                         w_hbm.at[pl.ds(soff, part)], sem)

    poi_dma.wait()

    def group(g):
        off = pl.multiple_of(g * 16, 16)
        up = uid_v[pl.ds(off, 16)] * D
        pp = pid_v[pl.ds(off, 16)] * D
        sp = sid_v[pl.ds(off, 16)] * D
        sn = nid_v[pl.ds(off, 16)] * D
        zero = jnp.zeros((16,), jnp.float32)
        ap = [zero] * 4
        an = [zero] * 4
        lane = lax.iota(jnp.int32, 16)
        for j in range(D):
            jr = (lane + j) & (D - 1)
            t = (plsc.load_gather(poi_v, [pp + jr])
                 + plsc.load_gather(usr_v, [up + jr]))
            ep = t - plsc.load_gather(poi_v, [sp + jr])
            en = t - plsc.load_gather(poi_v, [sn + jr])
            q = j & 3
            ap[q] = ap[q] + ep * ep
            an[q] = an[q] + en * en
        acc_p = (ap[0] + ap[1]) + (ap[2] + ap[3])
        acc_n = (an[0] + an[1]) + (an[2] + an[3])
        obj = obj_v[pl.ds(off, 16)] + _sqrt_nr(acc_n) - _sqrt_nr(acc_p)
        obj_v[pl.ds(off, 16)] = obj

    with jax.named_scope("main_loop"):
        plsc.parallel_loop(0, NG, 1, unroll=2)(group)
    obj_dma = pltpu.async_copy(obj_v, obj_hbm.at[pl.ds(base, C)], sem)

    @pl.when(sid < NS - 1)
    def _():
        pltpu.make_async_copy(wout_v, w_hbm.at[pl.ds(soff, full)],
                              sem).wait()

    @pl.when(sid == NS - 1)
    def _():
        pltpu.make_async_copy(wout_v.at[pl.ds(0, part)],
                              w_hbm.at[pl.ds(soff, part)], sem).wait()

    obj_dma.wait()


@jax.jit
def kernel(user_id, prev_id, pos_id, neg_id, poi_weight, user_weight,
           user_global_weight, poi_bias_weight):
    mesh = plsc.VectorSubcoreMesh(core_axis_name="c", subcore_axis_name="s")
    prog = pl.kernel(
        _body,
        out_type=(
            jax.ShapeDtypeStruct((B,), jnp.float32),
            jax.ShapeDtypeStruct((NP * D,), jnp.float32),
        ),
        mesh=mesh,
        compiler_params=pltpu.CompilerParams(
            needs_layout_passes=False,
            disable_bounds_checks=True,
        ),
        scratch_types=[
            pltpu.VMEM((NP * D,), jnp.float32),
            pltpu.VMEM((NU * D,), jnp.float32),
            pltpu.VMEM((NP,), jnp.float32),
            pltpu.VMEM((D,), jnp.float32),
            pltpu.VMEM((C,), jnp.int32),
            pltpu.VMEM((C,), jnp.int32),
            pltpu.VMEM((C,), jnp.int32),
            pltpu.VMEM((C,), jnp.int32),
            pltpu.VMEM((C,), jnp.float32),
            pltpu.VMEM((RPT * D,), jnp.float32),
            pltpu.VMEM_SHARED((NP * D,), jnp.float32),
            pltpu.SemaphoreType.DMA,
            pltpu.SemaphoreType.DMA,
        ],
    )
    obj, w = prog(
        user_id.astype(jnp.int32),
        prev_id.astype(jnp.int32),
        pos_id.astype(jnp.int32),
        neg_id.astype(jnp.int32),
        poi_weight.reshape(NP * D),
        user_weight.reshape(NU * D),
        user_global_weight.reshape(D),
        poi_bias_weight.reshape(NP),
    )
    return obj, w.reshape(NP, D)


# speedup vs baseline: 1.2616x; 1.0145x over previous
import jax
import jax.numpy as jnp
from jax import lax
from jax.experimental import pallas as pl
from jax.experimental.pallas import tpu as pltpu
from jax.experimental.pallas import tpu_sc as plsc

B = 16384
D = 64
NP = 1000
NU = 100
NC = 2
NS = 16
NW = NC * NS
C = B // NW
RPT = 64
NG = C // 16


def _rsqrt_nr(s):
    i = plsc.bitcast(s, jnp.int32)
    i = jnp.int32(0x5F3759DF) - (i >> 1)
    y = plsc.bitcast(i, jnp.float32)
    for _ in range(3):
        y = y * (1.5 - ((0.5 * s) * y) * y)
    return y


def _sqrt_nr(s):
    return s * _rsqrt_nr(s)


def _body(user_hbm, prev_hbm, pos_hbm, neg_hbm, poi_hbm, usr_hbm, g_hbm,
          bias_hbm, obj_hbm, w_hbm,
          poi_v, usr_v, bias_v, g_v, uid_v, pid_v, sid_v, nid_v,
          obj_v, wout_v, poi_sh, sem, sem2):
    wid = lax.axis_index("s") * NC + lax.axis_index("c")
    base = wid * C

    with jax.named_scope("stage_dma"):
        sid = lax.axis_index("s")
        soff = pl.multiple_of(sid * RPT * D, 8)
        dmas = [
            pltpu.async_copy(usr_hbm, usr_v, sem),
            pltpu.async_copy(bias_hbm, bias_v, sem),
            pltpu.async_copy(g_hbm, g_v, sem),
            pltpu.async_copy(user_hbm.at[pl.ds(base, C)], uid_v, sem),
            pltpu.async_copy(prev_hbm.at[pl.ds(base, C)], pid_v, sem),
            pltpu.async_copy(pos_hbm.at[pl.ds(base, C)], sid_v, sem),
            pltpu.async_copy(neg_hbm.at[pl.ds(base, C)], nid_v, sem),
        ]
        full = RPT * D
        part = (NP - (NS - 1) * RPT) * D

        @pl.when(sid < NS - 1)
        def _():
            pltpu.sync_copy(poi_hbm.at[pl.ds(soff, full)],
                            poi_v.at[pl.ds(soff, full)])
            pltpu.sync_copy(poi_v.at[pl.ds(soff, full)],
                            poi_sh.at[pl.ds(soff, full)])

        @pl.when(sid == NS - 1)
        def _():
            pltpu.sync_copy(poi_hbm.at[pl.ds(soff, part)],
                            poi_v.at[pl.ds(soff, part)])
            pltpu.sync_copy(poi_v.at[pl.ds(soff, part)],
                            poi_sh.at[pl.ds(soff, part)])

        plsc.subcore_barrier()
        poi_dma = pltpu.async_copy(poi_sh, poi_v, sem2)
        for d in dmas:
            d.wait()

    g0 = g_v[pl.ds(0, 16)]
    g1 = g_v[pl.ds(16, 16)]
    g2v = g_v[pl.ds(32, 16)]
    g3 = g_v[pl.ds(48, 16)]

    def fold(r, gs):
        ro = pl.multiple_of(r * D, 16)
        for q in range(4):
            off_q = ro + q * 16
            usr_v[pl.ds(off_q, 16)] = usr_v[pl.ds(off_q, 16)] + gs[q]
        return gs

    lax.fori_loop(0, NU, fold, (g0, g1, g2v, g3))

    base_r = sid * RPT
    iota = lax.iota(jnp.int32, 16)
    for g2 in range(RPT // 16):
        ridx = jnp.minimum(base_r + g2 * 16 + iota, NP - 1) * D

        def nsum(j, acc):
            jr = (iota + j) & (D - 1)
            v = plsc.load_gather(poi_v, [ridx + jr])
            return acc + v * v

        s = lax.fori_loop(0, D, nsum, jnp.zeros((16,), jnp.float32))
        scale = jnp.minimum(1.0, _rsqrt_nr(s))
        lrow = (g2 * 16 + iota) * D

        def nwrite(j, carry):
            jr = (iota + j) & (D - 1)
            v = plsc.load_gather(poi_v, [ridx + jr])
            plsc.store_scatter(wout_v, [lrow + jr], v * scale)
            return carry

        lax.fori_loop(0, D, nwrite, None)

    def biasg(g, carry):
        off = pl.multiple_of(g * 16, 16)
        bp = plsc.load_gather(bias_v, [sid_v[pl.ds(off, 16)]])
        bn = plsc.load_gather(bias_v, [nid_v[pl.ds(off, 16)]])
        obj_v[pl.ds(off, 16)] = bp - bn
        return carry

    lax.fori_loop(0, NG, biasg, None)

    @pl.when(sid < NS - 1)
    def _():
        pltpu.async_copy(wout_v, w_hbm.at[pl.ds(soff, full)], sem)

    @pl.when(sid == NS - 1)
    def _():
        pltpu.async_copy(wout_v.at[pl.ds(0, part)],
                         w_hbm.at[pl.ds(soff, part)], sem)

    poi_dma.wait()

    def group(g):
        off = pl.multiple_of(g * 16, 16)
        up = uid_v[pl.ds(off, 16)] * D
        pp = pid_v[pl.ds(off, 16)] * D
        sp = sid_v[pl.ds(off, 16)] * D
        sn = nid_v[pl.ds(off, 16)] * D
        zero = jnp.zeros((16,), jnp.float32)
        ap = [zero] * 4
        an = [zero] * 4
        lane = lax.iota(jnp.int32, 16)
        for j in range(D):
            jr = (lane + j) & (D - 1)
            t = (plsc.load_gather(poi_v, [pp + jr])
                 + plsc.load_gather(usr_v, [up + jr]))
            ep = t - plsc.load_gather(poi_v, [sp + jr])
            en = t - plsc.load_gather(poi_v, [sn + jr])
            q = j & 3
            ap[q] = ap[q] + ep * ep
            an[q] = an[q] + en * en
        acc_p = (ap[0] + ap[1]) + (ap[2] + ap[3])
        acc_n = (an[0] + an[1]) + (an[2] + an[3])
        obj = obj_v[pl.ds(off, 16)] + _sqrt_nr(acc_n) - _sqrt_nr(acc_p)
        obj_v[pl.ds(off, 16)] = obj

    with jax.named_scope("main_loop"):
        plsc.parallel_loop(0, NG, 1, unroll=2)(group)
    obj_dma = pltpu.async_copy(obj_v, obj_hbm.at[pl.ds(base, C)], sem)

    @pl.when(sid < NS - 1)
    def _():
        pltpu.make_async_copy(wout_v, w_hbm.at[pl.ds(soff, full)],
                              sem).wait()

    @pl.when(sid == NS - 1)
    def _():
        pltpu.make_async_copy(wout_v.at[pl.ds(0, part)],
                              w_hbm.at[pl.ds(soff, part)], sem).wait()

    obj_dma.wait()


@jax.jit
def kernel(user_id, prev_id, pos_id, neg_id, poi_weight, user_weight,
           user_global_weight, poi_bias_weight):
    mesh = plsc.VectorSubcoreMesh(core_axis_name="c", subcore_axis_name="s")
    prog = pl.kernel(
        _body,
        out_type=(
            jax.ShapeDtypeStruct((B,), jnp.float32),
            jax.ShapeDtypeStruct((NP * D,), jnp.float32),
        ),
        mesh=mesh,
        compiler_params=pltpu.CompilerParams(
            needs_layout_passes=False,
            disable_bounds_checks=True,
        ),
        scratch_types=[
            pltpu.VMEM((NP * D,), jnp.float32),
            pltpu.VMEM((NU * D,), jnp.float32),
            pltpu.VMEM((NP,), jnp.float32),
            pltpu.VMEM((D,), jnp.float32),
            pltpu.VMEM((C,), jnp.int32),
            pltpu.VMEM((C,), jnp.int32),
            pltpu.VMEM((C,), jnp.int32),
            pltpu.VMEM((C,), jnp.int32),
            pltpu.VMEM((C,), jnp.float32),
            pltpu.VMEM((RPT * D,), jnp.float32),
            pltpu.VMEM_SHARED((NP * D,), jnp.float32),
            pltpu.SemaphoreType.DMA,
            pltpu.SemaphoreType.DMA,
        ],
    )
    obj, w = prog(
        user_id.astype(jnp.int32),
        prev_id.astype(jnp.int32),
        pos_id.astype(jnp.int32),
        neg_id.astype(jnp.int32),
        poi_weight.reshape(NP * D),
        user_weight.reshape(NU * D),
        user_global_weight.reshape(D),
        poi_bias_weight.reshape(NP),
    )
    return obj, w.reshape(NP, D)
